# baseline (speedup 1.0000x reference)
import functools

import jax
import jax.numpy as jnp
from jax import lax
from jax.experimental import pallas as pl
from jax.experimental.pallas import tpu as pltpu
from jax.experimental.pallas import tpu_sc as plsc

N = 10000
D = 64
U = 4096
INV30 = 1.0 / 30.0

NC, NS = 2, 16
NW = NC * NS
CHUNK = U // NW

BLK_A = 256


def _winner_body(src_col_ref, src_row_ref, ts_row_ref, msg_ref,
                 w_ref, tsw_ref, msgp_ref):
    a = src_col_ref[...]
    b = src_row_ref[...]
    eq = a == b
    j = lax.broadcasted_iota(jnp.int32, (BLK_A, U), 1)
    w = jnp.max(jnp.where(eq, j, -1), axis=1, keepdims=True)
    w_ref[...] = w
    at_w = j == w
    ts_row = ts_row_ref[...]
    tsw_ref[...] = jnp.max(jnp.where(at_w, ts_row, -jnp.inf), axis=1,
                           keepdims=True)
    msg = msg_ref[...]
    msgp_ref[...] = jnp.concatenate(
        [msg, jnp.zeros((BLK_A, 2 * D - (D + 1)), jnp.float32)], axis=1)


def _make_winner(interpret=False):
    return pl.pallas_call(
        _winner_body,
        grid=(U // BLK_A,),
        in_specs=[
            pl.BlockSpec((BLK_A, 1), lambda i: (i, 0)),
            pl.BlockSpec((1, U), lambda i: (0, 0)),
            pl.BlockSpec((1, U), lambda i: (0, 0)),
            pl.BlockSpec((BLK_A, D + 1), lambda i: (i, 0)),
        ],
        out_specs=[
            pl.BlockSpec((BLK_A, 1), lambda i: (i, 0)),
            pl.BlockSpec((BLK_A, 1), lambda i: (i, 0)),
            pl.BlockSpec((BLK_A, 2 * D), lambda i: (i, 0)),
        ],
        out_shape=[
            jax.ShapeDtypeStruct((U, 1), jnp.int32),
            jax.ShapeDtypeStruct((U, 1), jnp.float32),
            jax.ShapeDtypeStruct((U, 2 * D), jnp.float32),
        ],
        interpret=interpret,
    )


def _gather_body(src_h, w_h, mem_h, statp_h, msgp_h, lu_h,
                 rows_o, statp_o, msgw_o, lug_o,
                 idx_v, widx_v, pidx_v, rows_v, stat_v, msg_v,
                 lu_sp, lu_all, lug_v, sem, sem2):
    s = lax.axis_index("s")
    wid = s * NC + lax.axis_index("c")
    base = wid * CHUNK
    ci = pltpu.async_copy(src_h.at[pl.ds(base, CHUNK)], idx_v, sem)
    cw = pltpu.async_copy(w_h.at[pl.ds(base, CHUNK)], widx_v, sem)

    @pl.when(s == 0)
    def _():
        pltpu.sync_copy(lu_h, lu_sp)

    plsc.subcore_barrier()
    cl = pltpu.async_copy(lu_sp, lu_all, sem2)
    ci.wait()
    cw.wait()
    c1 = pltpu.async_copy(mem_h.at[idx_v], rows_v, sem)
    for k in range(CHUNK // 16):
        iv = idx_v[pl.ds(16 * k, 16)]
        pidx_v[pl.ds(16 * k, 16)] = lax.shift_right_logical(iv, 1)
    c2 = pltpu.async_copy(statp_h.at[pidx_v], stat_v, sem)
    c3 = pltpu.async_copy(msgp_h.at[widx_v], msg_v, sem)
    cl.wait()
    for k in range(CHUNK // 16):
        iv = idx_v[pl.ds(16 * k, 16)]
        lug_v[pl.ds(16 * k, 16)] = plsc.load_gather(lu_all, [iv])
    c1.wait()
    c2.wait()
    c3.wait()
    o1 = pltpu.async_copy(rows_v, rows_o.at[pl.ds(base, CHUNK)], sem)
    o2 = pltpu.async_copy(stat_v, statp_o.at[pl.ds(base, CHUNK)], sem)
    o3 = pltpu.async_copy(msg_v, msgw_o.at[pl.ds(base, CHUNK)], sem)
    o4 = pltpu.async_copy(lug_v, lug_o.at[pl.ds(base, CHUNK)], sem)
    o1.wait()
    o2.wait()
    o3.wait()
    o4.wait()


def _make_gather(interpret=False):
    mesh = plsc.VectorSubcoreMesh(
        core_axis_name="c", subcore_axis_name="s",
        num_cores=NC, num_subcores=NS)
    return functools.partial(
        pl.kernel,
        out_type=(
            jax.ShapeDtypeStruct((U, 2 * D), jnp.float32),
            jax.ShapeDtypeStruct((U, 2 * D), jnp.float32),
            jax.ShapeDtypeStruct((U, 2 * D), jnp.float32),
            jax.ShapeDtypeStruct((U,), jnp.float32),
        ),
        mesh=mesh,
        compiler_params=pltpu.CompilerParams(needs_layout_passes=False),
        scratch_types=[
            pltpu.VMEM((CHUNK,), jnp.int32),
            pltpu.VMEM((CHUNK,), jnp.int32),
            pltpu.VMEM((CHUNK,), jnp.int32),
            pltpu.VMEM((CHUNK, 2 * D), jnp.float32),
            pltpu.VMEM((CHUNK, 2 * D), jnp.float32),
            pltpu.VMEM((CHUNK, 2 * D), jnp.float32),
            pltpu.VMEM_SHARED((N,), jnp.float32),
            pltpu.VMEM((N,), jnp.float32),
            pltpu.VMEM((CHUNK,), jnp.float32),
            pltpu.SemaphoreType.DMA,
            pltpu.SemaphoreType.DMA,
        ],
        interpret=interpret,
    )(_gather_body)


def _mlp(x, w1, b1, w2, b2):
    h = jnp.dot(x, w1, preferred_element_type=jnp.float32) + b1
    h = jnp.where(h >= 0, h, 0.01 * h)
    h = jnp.dot(h, w2, preferred_element_type=jnp.float32) + b2
    return jnp.where(h >= 0, h, 0.01 * h)


BLK_N = 1000
BLK_F = 1024
GRID_C = N // BLK_N + U // BLK_F


def _dense_fix_body(mem_ref, lu_ref, stat_ref,
                    rows_ref, lug_ref, tsw_ref, msgw_ref, statp_ref, src_ref,
                    w1_ref, b1_ref, w2_ref, b2_ref, el_ref, nt_ref,
                    out_ref, fix_ref):
    w1, b1, w2, b2 = w1_ref[...], b1_ref[...], w2_ref[...], b2_ref[...]
    el = el_ref[...]
    nt = nt_ref[...]
    h = _mlp(mem_ref[...], w1, b1, w2, b2)
    outdec = jnp.exp((lu_ref[...] - nt) * INV30)
    res = el * stat_ref[...] + (1.0 - el) * outdec * h
    out_ref[...] = jnp.concatenate([res, res], axis=1)
    rows = rows_ref[...]
    tsw = tsw_ref[...]
    dec = jnp.exp((lug_ref[...] - tsw) * INV30)
    msg = msgw_ref[...]
    x = rows * dec
    upd = jnp.concatenate(
        [x[:, :D] + msg[:, :D], x[:, D:] + msg[:, D:D + 1]], axis=1)
    hf = _mlp(upd, w1, b1, w2, b2)
    outdec_f = jnp.exp((tsw - nt) * INV30)
    statp = statp_ref[...]
    odd = (src_ref[...] & 1) == 1
    statg = jnp.where(odd, statp[:, D:], statp[:, :D])
    fix = el * statg + (1.0 - el) * outdec_f * hf
    fix_ref[...] = jnp.concatenate([fix, fix], axis=1)


def _make_dense_fix(interpret=False):
    nd = N // BLK_N
    dmap = lambda i: (jnp.minimum(i, nd - 1), 0)
    fmap = lambda i: (jnp.maximum(i - nd, 0), 0)
    const = lambda i: (0, 0)
    return pl.pallas_call(
        _dense_fix_body,
        grid=(GRID_C,),
        in_specs=[
            pl.BlockSpec((BLK_N, 2 * D), dmap),
            pl.BlockSpec((BLK_N, 1), dmap),
            pl.BlockSpec((BLK_N, D), dmap),
            pl.BlockSpec((BLK_F, 2 * D), fmap),
            pl.BlockSpec((BLK_F, 1), fmap),
            pl.BlockSpec((BLK_F, 1), fmap),
            pl.BlockSpec((BLK_F, 2 * D), fmap),
            pl.BlockSpec((BLK_F, 2 * D), fmap),
            pl.BlockSpec((BLK_F, 1), fmap),
            pl.BlockSpec((2 * D, D), const),
            pl.BlockSpec((1, D), const),
            pl.BlockSpec((D, D), const),
            pl.BlockSpec((1, D), const),
            pl.BlockSpec((1, 1), const),
            pl.BlockSpec((1, 1), const),
        ],
        out_specs=[
            pl.BlockSpec((BLK_N, 2 * D), dmap),
            pl.BlockSpec((BLK_F, 2 * D), fmap),
        ],
        out_shape=[
            jax.ShapeDtypeStruct((N, 2 * D), jnp.float32),
            jax.ShapeDtypeStruct((U, 2 * D), jnp.float32),
        ],
        interpret=interpret,
    )


def _scatter_body(fixw_h, sidx_h, final_ref, fix_v, idx_v, sem):
    wid = lax.axis_index("s") * NC + lax.axis_index("c")
    pltpu.sync_copy(fixw_h.at[pl.ds(wid * CHUNK, CHUNK)], fix_v)
    pltpu.sync_copy(sidx_h.at[wid], idx_v)
    pltpu.async_copy(fix_v, final_ref.at[idx_v.at[0]], sem).wait()


def _make_scatter(interpret=False):
    mesh = plsc.VectorSubcoreMesh(
        core_axis_name="c", subcore_axis_name="s",
        num_cores=NC, num_subcores=NS)
    return functools.partial(
        pl.kernel,
        out_type=(),
        mesh=mesh,
        scratch_types=[
            pltpu.VMEM((CHUNK, 2 * D), jnp.float32),
            pltpu.VMEM((1, 128), jnp.int32),
            pltpu.SemaphoreType.DMA,
        ],
        interpret=interpret,
    )(_scatter_body)


_winner = _make_winner()
_gather = _make_gather()
_dense_fix = _make_dense_fix()
_scatter = _make_scatter()


def kernel(source_nodes, now_time, unique_sources, unique_messages,
           unique_timestamps, memory, last_update, W1, b1, W2, b2,
           static_emb, e_lamb):
    src = unique_sources.astype(jnp.int32)
    ts = unique_timestamps.astype(jnp.float32)
    nt = jnp.asarray(now_time, jnp.float32).reshape(1, 1)
    el = e_lamb.reshape(1, 1)
    lu2d = last_update.reshape(N, 1)
    b1r = b1.reshape(1, D)
    b2r = b2.reshape(1, D)
    statp_view = static_emb.reshape(N // 2, 2 * D)

    w, tsw, msgp = _winner(src.reshape(U, 1), src.reshape(1, U),
                           ts.reshape(1, U), unique_messages)
    rows_g, statp_g, msgw_g, lug = _gather(
        src, w.reshape(U), memory, statp_view, msgp, last_update)
    dense_w, fixw = _dense_fix(
        memory, lu2d, static_emb,
        rows_g, lug.reshape(U, 1), tsw, msgw_g, statp_g, src.reshape(U, 1),
        W1, b1r, W2, b2r, el, nt)
    final_ref = jax.new_ref(dense_w)
    _scatter(fixw, src.reshape(NW, 1, 128), final_ref)
    return final_ref[...][:, :D]


# speedup vs baseline: 1.0360x; 1.0360x over previous
import functools

import jax
import jax.numpy as jnp
from jax import lax
from jax.experimental import pallas as pl
from jax.experimental.pallas import tpu as pltpu
from jax.experimental.pallas import tpu_sc as plsc

N = 10000
D = 64
U = 4096
INV30 = 1.0 / 30.0

NC, NS = 2, 16
NW = NC * NS
CHUNK = U // NW

BLK_A = 256


def _winner_body(src_col_ref, src_row_ref, ts_row_ref, msg_ref,
                 w_ref, tsw_ref, msgp_ref):
    a = src_col_ref[...]
    b = src_row_ref[...]
    eq = a == b
    j = lax.broadcasted_iota(jnp.int32, (BLK_A, U), 1)
    w = jnp.max(jnp.where(eq, j, -1), axis=1, keepdims=True)
    w_ref[...] = w
    at_w = j == w
    ts_row = ts_row_ref[...]
    tsw_ref[...] = jnp.max(jnp.where(at_w, ts_row, -jnp.inf), axis=1,
                           keepdims=True)
    msg = msg_ref[...]
    msgp_ref[...] = jnp.concatenate(
        [msg, jnp.zeros((BLK_A, 2 * D - (D + 1)), jnp.float32)], axis=1)


def _make_winner(interpret=False):
    return pl.pallas_call(
        _winner_body,
        grid=(U // BLK_A,),
        in_specs=[
            pl.BlockSpec((BLK_A, 1), lambda i: (i, 0)),
            pl.BlockSpec((1, U), lambda i: (0, 0)),
            pl.BlockSpec((1, U), lambda i: (0, 0)),
            pl.BlockSpec((BLK_A, D + 1), lambda i: (i, 0)),
        ],
        out_specs=[
            pl.BlockSpec((BLK_A, 1), lambda i: (i, 0)),
            pl.BlockSpec((BLK_A, 1), lambda i: (i, 0)),
            pl.BlockSpec((BLK_A, 2 * D), lambda i: (i, 0)),
        ],
        out_shape=[
            jax.ShapeDtypeStruct((U, 1), jnp.int32),
            jax.ShapeDtypeStruct((U, 1), jnp.float32),
            jax.ShapeDtypeStruct((U, 2 * D), jnp.float32),
        ],
        interpret=interpret,
    )


def _gather_body(src_h, w_h, mem_h, statp_h, msgp_h, lu_h,
                 rows_o, statp_o, msgw_o, lug_o,
                 idx_v, widx_v, pidx_v, rows_v, stat_v, msg_v,
                 lu_all, lug_v, sem):
    wid = lax.axis_index("s") * NC + lax.axis_index("c")
    base = wid * CHUNK
    pltpu.sync_copy(src_h.at[pl.ds(base, CHUNK)], idx_v)
    pltpu.sync_copy(w_h.at[pl.ds(base, CHUNK)], widx_v)
    pltpu.sync_copy(lu_h, lu_all)
    for k in range(CHUNK // 16):
        iv = idx_v[pl.ds(16 * k, 16)]
        pidx_v[pl.ds(16 * k, 16)] = lax.shift_right_logical(iv, 1)
        lug_v[pl.ds(16 * k, 16)] = plsc.load_gather(lu_all, [iv])
    c1 = pltpu.async_copy(mem_h.at[idx_v], rows_v, sem)
    c2 = pltpu.async_copy(statp_h.at[pidx_v], stat_v, sem)
    c3 = pltpu.async_copy(msgp_h.at[widx_v], msg_v, sem)
    c1.wait()
    c2.wait()
    c3.wait()
    pltpu.sync_copy(rows_v, rows_o.at[pl.ds(base, CHUNK)])
    pltpu.sync_copy(stat_v, statp_o.at[pl.ds(base, CHUNK)])
    pltpu.sync_copy(msg_v, msgw_o.at[pl.ds(base, CHUNK)])
    pltpu.sync_copy(lug_v, lug_o.at[pl.ds(base, CHUNK)])


def _make_gather(interpret=False):
    mesh = plsc.VectorSubcoreMesh(
        core_axis_name="c", subcore_axis_name="s",
        num_cores=NC, num_subcores=NS)
    return functools.partial(
        pl.kernel,
        out_type=(
            jax.ShapeDtypeStruct((U, 2 * D), jnp.float32),
            jax.ShapeDtypeStruct((U, 2 * D), jnp.float32),
            jax.ShapeDtypeStruct((U, 2 * D), jnp.float32),
            jax.ShapeDtypeStruct((U,), jnp.float32),
        ),
        mesh=mesh,
        compiler_params=pltpu.CompilerParams(needs_layout_passes=False),
        scratch_types=[
            pltpu.VMEM((CHUNK,), jnp.int32),
            pltpu.VMEM((CHUNK,), jnp.int32),
            pltpu.VMEM((CHUNK,), jnp.int32),
            pltpu.VMEM((CHUNK, 2 * D), jnp.float32),
            pltpu.VMEM((CHUNK, 2 * D), jnp.float32),
            pltpu.VMEM((CHUNK, 2 * D), jnp.float32),
            pltpu.VMEM((N,), jnp.float32),
            pltpu.VMEM((CHUNK,), jnp.float32),
            pltpu.SemaphoreType.DMA,
        ],
        interpret=interpret,
    )(_gather_body)


def _mlp(x, w1, b1, w2, b2):
    h = jnp.dot(x, w1, preferred_element_type=jnp.float32) + b1
    h = jnp.where(h >= 0, h, 0.01 * h)
    h = jnp.dot(h, w2, preferred_element_type=jnp.float32) + b2
    return jnp.where(h >= 0, h, 0.01 * h)


BLK_F = 1024


def _fix_body(rows_ref, lug_ref, tsw_ref, msgw_ref, statp_ref, src_ref,
              w1_ref, b1_ref, w2_ref, b2_ref, el_ref, nt_ref, fix_ref):
    rows = rows_ref[...]
    tsw = tsw_ref[...]
    dec = jnp.exp((lug_ref[...] - tsw) * INV30)
    msg = msgw_ref[...]
    x = rows * dec
    upd = jnp.concatenate(
        [x[:, :D] + msg[:, :D], x[:, D:] + msg[:, D:D + 1]], axis=1)
    h = _mlp(upd, w1_ref[...], b1_ref[...], w2_ref[...], b2_ref[...])
    outdec = jnp.exp((tsw - nt_ref[...]) * INV30)
    statp = statp_ref[...]
    odd = (src_ref[...] & 1) == 1
    statg = jnp.where(odd, statp[:, D:], statp[:, :D])
    el = el_ref[...]
    fix = el * statg + (1.0 - el) * outdec * h
    fix_ref[...] = jnp.concatenate([fix, fix], axis=1)


def _make_fix(interpret=False):
    const = lambda i: (0, 0)
    return pl.pallas_call(
        _fix_body,
        grid=(U // BLK_F,),
        in_specs=[
            pl.BlockSpec((BLK_F, 2 * D), lambda i: (i, 0)),
            pl.BlockSpec((BLK_F, 1), lambda i: (i, 0)),
            pl.BlockSpec((BLK_F, 1), lambda i: (i, 0)),
            pl.BlockSpec((BLK_F, 2 * D), lambda i: (i, 0)),
            pl.BlockSpec((BLK_F, 2 * D), lambda i: (i, 0)),
            pl.BlockSpec((BLK_F, 1), lambda i: (i, 0)),
            pl.BlockSpec((2 * D, D), const),
            pl.BlockSpec((1, D), const),
            pl.BlockSpec((D, D), const),
            pl.BlockSpec((1, D), const),
            pl.BlockSpec((1, 1), const),
            pl.BlockSpec((1, 1), const),
        ],
        out_specs=pl.BlockSpec((BLK_F, 2 * D), lambda i: (i, 0)),
        out_shape=jax.ShapeDtypeStruct((U, 2 * D), jnp.float32),
        interpret=interpret,
    )


BLK_N = 1000


def _dense_body(mem_ref, lu_ref, stat_ref,
                w1_ref, b1_ref, w2_ref, b2_ref, el_ref, nt_ref, out_ref):
    h = _mlp(mem_ref[...], w1_ref[...], b1_ref[...], w2_ref[...], b2_ref[...])
    outdec = jnp.exp((lu_ref[...] - nt_ref[...]) * INV30)
    el = el_ref[...]
    res = el * stat_ref[...] + (1.0 - el) * outdec * h
    out_ref[...] = jnp.concatenate([res, res], axis=1)


def _make_dense(interpret=False):
    const = lambda i: (0, 0)
    return pl.pallas_call(
        _dense_body,
        grid=(N // BLK_N,),
        in_specs=[
            pl.BlockSpec((BLK_N, 2 * D), lambda i: (i, 0)),
            pl.BlockSpec((BLK_N, 1), lambda i: (i, 0)),
            pl.BlockSpec((BLK_N, D), lambda i: (i, 0)),
            pl.BlockSpec((2 * D, D), const),
            pl.BlockSpec((1, D), const),
            pl.BlockSpec((D, D), const),
            pl.BlockSpec((1, D), const),
            pl.BlockSpec((1, 1), const),
            pl.BlockSpec((1, 1), const),
        ],
        out_specs=pl.BlockSpec((BLK_N, 2 * D), lambda i: (i, 0)),
        out_shape=jax.ShapeDtypeStruct((N, 2 * D), jnp.float32),
        interpret=interpret,
    )


def _scatter_body(fixw_h, sidx_h, final_ref, fix_v, idx_v, sem):
    wid = lax.axis_index("s") * NC + lax.axis_index("c")
    pltpu.sync_copy(fixw_h.at[pl.ds(wid * CHUNK, CHUNK)], fix_v)
    pltpu.sync_copy(sidx_h.at[wid], idx_v)
    pltpu.async_copy(fix_v, final_ref.at[idx_v.at[0]], sem).wait()


def _make_scatter(interpret=False):
    mesh = plsc.VectorSubcoreMesh(
        core_axis_name="c", subcore_axis_name="s",
        num_cores=NC, num_subcores=NS)
    return functools.partial(
        pl.kernel,
        out_type=(),
        mesh=mesh,
        scratch_types=[
            pltpu.VMEM((CHUNK, 2 * D), jnp.float32),
            pltpu.VMEM((1, 128), jnp.int32),
            pltpu.SemaphoreType.DMA,
        ],
        interpret=interpret,
    )(_scatter_body)


_winner = _make_winner()
_gather = _make_gather()
_fix = _make_fix()
_dense = _make_dense()
_scatter = _make_scatter()


def kernel(source_nodes, now_time, unique_sources, unique_messages,
           unique_timestamps, memory, last_update, W1, b1, W2, b2,
           static_emb, e_lamb):
    src = unique_sources.astype(jnp.int32)
    ts = unique_timestamps.astype(jnp.float32)
    nt = jnp.asarray(now_time, jnp.float32).reshape(1, 1)
    el = e_lamb.reshape(1, 1)
    lu2d = last_update.reshape(N, 1)
    b1r = b1.reshape(1, D)
    b2r = b2.reshape(1, D)
    statp_view = static_emb.reshape(N // 2, 2 * D)

    w, tsw, msgp = _winner(src.reshape(U, 1), src.reshape(1, U),
                           ts.reshape(1, U), unique_messages)
    rows_g, statp_g, msgw_g, lug = _gather(
        src, w.reshape(U), memory, statp_view, msgp, last_update)
    fixw = _fix(rows_g, lug.reshape(U, 1), tsw, msgw_g,
                statp_g, src.reshape(U, 1), W1, b1r, W2, b2r, el, nt)
    dense_w = _dense(memory, lu2d, static_emb, W1, b1r, W2, b2r, el, nt)
    final_ref = jax.new_ref(dense_w)
    _scatter(fixw, src.reshape(NW, 1, 128), final_ref)
    return final_ref[...][:, :D]


# speedup vs baseline: 1.0856x; 1.0479x over previous
import functools

import jax
import jax.numpy as jnp
from jax import lax
from jax.experimental import pallas as pl
from jax.experimental.pallas import tpu as pltpu
from jax.experimental.pallas import tpu_sc as plsc

N = 10000
D = 64
U = 4096
INV30 = 1.0 / 30.0

NC, NS = 2, 16
NW = NC * NS
CHUNK = U // NW

BLK_A = 256


def _winner_body(src_col_ref, src_row_ref, out_ref):
    a = src_col_ref[...]
    b = src_row_ref[...]
    eq = a == b
    j = lax.broadcasted_iota(jnp.int32, (BLK_A, U), 1)
    out_ref[...] = jnp.max(jnp.where(eq, j, -1), axis=1, keepdims=True)


def _make_winner(interpret=False):
    return pl.pallas_call(
        _winner_body,
        grid=(U // BLK_A,),
        in_specs=[
            pl.BlockSpec((BLK_A, 1), lambda i: (i, 0)),
            pl.BlockSpec((1, U), lambda i: (0, 0)),
        ],
        out_specs=pl.BlockSpec((BLK_A, 1), lambda i: (i, 0)),
        out_shape=jax.ShapeDtypeStruct((U, 1), jnp.int32),
        interpret=interpret,
    )


def _gather_body(src_h, w_h, mem_h, statp_h, msgp_h, lu_h, ts_h,
                 rows_o, statp_o, msgw_o, lug_o, tsw_o,
                 idx_v, widx_v, pidx_v, rows_v, stat_v, msg_v,
                 lu_all, ts_all, lug_v, tsw_v, sem):
    wid = lax.axis_index("s") * NC + lax.axis_index("c")
    base = wid * CHUNK
    pltpu.sync_copy(src_h.at[pl.ds(base, CHUNK)], idx_v)
    pltpu.sync_copy(w_h.at[pl.ds(base, CHUNK)], widx_v)
    pltpu.sync_copy(lu_h, lu_all)
    pltpu.sync_copy(ts_h, ts_all)
    for k in range(CHUNK // 16):
        iv = idx_v[pl.ds(16 * k, 16)]
        wv = widx_v[pl.ds(16 * k, 16)]
        pidx_v[pl.ds(16 * k, 16)] = lax.shift_right_logical(iv, 1)
        lug_v[pl.ds(16 * k, 16)] = plsc.load_gather(lu_all, [iv])
        tsw_v[pl.ds(16 * k, 16)] = plsc.load_gather(ts_all, [wv])
    c1 = pltpu.async_copy(mem_h.at[idx_v], rows_v, sem)
    c2 = pltpu.async_copy(statp_h.at[pidx_v], stat_v, sem)
    c3 = pltpu.async_copy(msgp_h.at[widx_v], msg_v, sem)
    c1.wait()
    c2.wait()
    c3.wait()
    pltpu.sync_copy(rows_v, rows_o.at[pl.ds(base, CHUNK)])
    pltpu.sync_copy(stat_v, statp_o.at[pl.ds(base, CHUNK)])
    pltpu.sync_copy(msg_v, msgw_o.at[pl.ds(base, CHUNK)])
    pltpu.sync_copy(lug_v, lug_o.at[pl.ds(base, CHUNK)])
    pltpu.sync_copy(tsw_v, tsw_o.at[pl.ds(base, CHUNK)])


def _make_gather(interpret=False):
    mesh = plsc.VectorSubcoreMesh(
        core_axis_name="c", subcore_axis_name="s",
        num_cores=NC, num_subcores=NS)
    return functools.partial(
        pl.kernel,
        out_type=(
            jax.ShapeDtypeStruct((U, 2 * D), jnp.float32),
            jax.ShapeDtypeStruct((U, 2 * D), jnp.float32),
            jax.ShapeDtypeStruct((U, 2 * D), jnp.float32),
            jax.ShapeDtypeStruct((U,), jnp.float32),
            jax.ShapeDtypeStruct((U,), jnp.float32),
        ),
        mesh=mesh,
        compiler_params=pltpu.CompilerParams(needs_layout_passes=False),
        scratch_types=[
            pltpu.VMEM((CHUNK,), jnp.int32),
            pltpu.VMEM((CHUNK,), jnp.int32),
            pltpu.VMEM((CHUNK,), jnp.int32),
            pltpu.VMEM((CHUNK, 2 * D), jnp.float32),
            pltpu.VMEM((CHUNK, 2 * D), jnp.float32),
            pltpu.VMEM((CHUNK, 2 * D), jnp.float32),
            pltpu.VMEM((N,), jnp.float32),
            pltpu.VMEM((U,), jnp.float32),
            pltpu.VMEM((CHUNK,), jnp.float32),
            pltpu.VMEM((CHUNK,), jnp.float32),
            pltpu.SemaphoreType.DMA,
        ],
        interpret=interpret,
    )(_gather_body)


def _mlp(x, w1, b1, w2, b2):
    h = jnp.dot(x, w1, preferred_element_type=jnp.float32) + b1
    h = jnp.where(h >= 0, h, 0.01 * h)
    h = jnp.dot(h, w2, preferred_element_type=jnp.float32) + b2
    return jnp.where(h >= 0, h, 0.01 * h)


BLK_F = 1024


def _fix_body(rows_ref, lug_ref, tsw_ref, msgw_ref, statp_ref, src_ref,
              w1_ref, b1_ref, w2_ref, b2_ref, el_ref, nt_ref, fix_ref):
    rows = rows_ref[...]
    tsw = tsw_ref[...]
    dec = jnp.exp((lug_ref[...] - tsw) * INV30)
    msg = msgw_ref[...]
    x = rows * dec
    upd = jnp.concatenate(
        [x[:, :D] + msg[:, :D], x[:, D:] + msg[:, D:D + 1]], axis=1)
    h = _mlp(upd, w1_ref[...], b1_ref[...], w2_ref[...], b2_ref[...])
    outdec = jnp.exp((tsw - nt_ref[...]) * INV30)
    statp = statp_ref[...]
    odd = (src_ref[...] & 1) == 1
    statg = jnp.where(odd, statp[:, D:], statp[:, :D])
    el = el_ref[...]
    fix = el * statg + (1.0 - el) * outdec * h
    fix_ref[...] = jnp.concatenate([fix, fix], axis=1)


def _make_fix(interpret=False):
    const = lambda i: (0, 0)
    return pl.pallas_call(
        _fix_body,
        grid=(U // BLK_F,),
        in_specs=[
            pl.BlockSpec((BLK_F, 2 * D), lambda i: (i, 0)),
            pl.BlockSpec((BLK_F, 1), lambda i: (i, 0)),
            pl.BlockSpec((BLK_F, 1), lambda i: (i, 0)),
            pl.BlockSpec((BLK_F, 2 * D), lambda i: (i, 0)),
            pl.BlockSpec((BLK_F, 2 * D), lambda i: (i, 0)),
            pl.BlockSpec((BLK_F, 1), lambda i: (i, 0)),
            pl.BlockSpec((2 * D, D), const),
            pl.BlockSpec((1, D), const),
            pl.BlockSpec((D, D), const),
            pl.BlockSpec((1, D), const),
            pl.BlockSpec((1, 1), const),
            pl.BlockSpec((1, 1), const),
        ],
        out_specs=pl.BlockSpec((BLK_F, 2 * D), lambda i: (i, 0)),
        out_shape=jax.ShapeDtypeStruct((U, 2 * D), jnp.float32),
        interpret=interpret,
    )


BLK_N = 1000


def _dense_body(mem_ref, lu_ref, stat_ref,
                w1_ref, b1_ref, w2_ref, b2_ref, el_ref, nt_ref, out_ref):
    h = _mlp(mem_ref[...], w1_ref[...], b1_ref[...], w2_ref[...], b2_ref[...])
    outdec = jnp.exp((lu_ref[...] - nt_ref[...]) * INV30)
    el = el_ref[...]
    res = el * stat_ref[...] + (1.0 - el) * outdec * h
    out_ref[...] = jnp.concatenate([res, res], axis=1)


def _make_dense(interpret=False):
    const = lambda i: (0, 0)
    return pl.pallas_call(
        _dense_body,
        grid=(N // BLK_N,),
        in_specs=[
            pl.BlockSpec((BLK_N, 2 * D), lambda i: (i, 0)),
            pl.BlockSpec((BLK_N, 1), lambda i: (i, 0)),
            pl.BlockSpec((BLK_N, D), lambda i: (i, 0)),
            pl.BlockSpec((2 * D, D), const),
            pl.BlockSpec((1, D), const),
            pl.BlockSpec((D, D), const),
            pl.BlockSpec((1, D), const),
            pl.BlockSpec((1, 1), const),
            pl.BlockSpec((1, 1), const),
        ],
        out_specs=pl.BlockSpec((BLK_N, 2 * D), lambda i: (i, 0)),
        out_shape=jax.ShapeDtypeStruct((N, 2 * D), jnp.float32),
        interpret=interpret,
    )


def _scatter_body(fixw_h, sidx_h, final_ref, fix_v, idx_v, sem):
    wid = lax.axis_index("s") * NC + lax.axis_index("c")
    pltpu.sync_copy(fixw_h.at[pl.ds(wid * CHUNK, CHUNK)], fix_v)
    pltpu.sync_copy(sidx_h.at[wid], idx_v)
    pltpu.async_copy(fix_v, final_ref.at[idx_v.at[0]], sem).wait()


def _make_scatter(interpret=False):
    mesh = plsc.VectorSubcoreMesh(
        core_axis_name="c", subcore_axis_name="s",
        num_cores=NC, num_subcores=NS)
    return functools.partial(
        pl.kernel,
        out_type=(),
        mesh=mesh,
        scratch_types=[
            pltpu.VMEM((CHUNK, 2 * D), jnp.float32),
            pltpu.VMEM((1, 128), jnp.int32),
            pltpu.SemaphoreType.DMA,
        ],
        interpret=interpret,
    )(_scatter_body)


_winner = _make_winner()
_gather = _make_gather()
_fix = _make_fix()
_dense = _make_dense()
_scatter = _make_scatter()


def kernel(source_nodes, now_time, unique_sources, unique_messages,
           unique_timestamps, memory, last_update, W1, b1, W2, b2,
           static_emb, e_lamb):
    src = unique_sources.astype(jnp.int32)
    ts = unique_timestamps.astype(jnp.float32)
    nt = jnp.asarray(now_time, jnp.float32).reshape(1, 1)
    el = e_lamb.reshape(1, 1)
    lu2d = last_update.reshape(N, 1)
    b1r = b1.reshape(1, D)
    b2r = b2.reshape(1, D)
    statp_view = static_emb.reshape(N // 2, 2 * D)
    msgp = jnp.pad(unique_messages, ((0, 0), (0, 2 * D - (D + 1))))

    w = _winner(src.reshape(U, 1), src.reshape(1, U))
    rows_g, statp_g, msgw_g, lug, tsw = _gather(
        src, w.reshape(U), memory, statp_view, msgp, last_update, ts)
    fixw = _fix(rows_g, lug.reshape(U, 1), tsw.reshape(U, 1), msgw_g,
                statp_g, src.reshape(U, 1), W1, b1r, W2, b2r, el, nt)
    dense_w = _dense(memory, lu2d, static_emb, W1, b1r, W2, b2r, el, nt)
    final_ref = jax.new_ref(dense_w)
    _scatter(fixw, src.reshape(NW, 1, 128), final_ref)
    return final_ref[...][:, :D]


# speedup vs baseline: 1.0864x; 1.0007x over previous
import functools

import jax
import jax.numpy as jnp
from jax import lax
from jax.experimental import pallas as pl
from jax.experimental.pallas import tpu as pltpu
from jax.experimental.pallas import tpu_sc as plsc

N = 10000
D = 64
U = 4096
INV30 = 1.0 / 30.0

NC, NS = 2, 16
NW = NC * NS
CHUNK = U // NW

BLK_A = 256


def _winner_body(src_col_ref, src_row_ref, out_ref):
    a = src_col_ref[...]
    b = src_row_ref[...]
    eq = a == b
    j = lax.broadcasted_iota(jnp.int32, (BLK_A, U), 1)
    out_ref[...] = jnp.max(jnp.where(eq, j, -1), axis=1, keepdims=True)


def _make_winner(interpret=False):
    return pl.pallas_call(
        _winner_body,
        grid=(U // BLK_A,),
        in_specs=[
            pl.BlockSpec((BLK_A, 1), lambda i: (i, 0)),
            pl.BlockSpec((1, U), lambda i: (0, 0)),
        ],
        out_specs=pl.BlockSpec((BLK_A, 1), lambda i: (i, 0)),
        out_shape=jax.ShapeDtypeStruct((U, 1), jnp.int32),
        interpret=interpret,
    )


def _gather_body(src_h, w_h, mem_h, statp_h, msgp_h, lu_h, ts_h,
                 rows_o, statp_o, msgw_o, lug_o, tsw_o,
                 idx_v, widx_v, pidx_v, rows_v, stat_v, msg_v,
                 lu_all, ts_all, lug_v, tsw_v, sem):
    wid = lax.axis_index("s") * NC + lax.axis_index("c")
    base = wid * CHUNK
    i1 = pltpu.async_copy(src_h.at[pl.ds(base, CHUNK)], idx_v, sem)
    i2 = pltpu.async_copy(w_h.at[pl.ds(base, CHUNK)], widx_v, sem)
    i3 = pltpu.async_copy(lu_h, lu_all, sem)
    i4 = pltpu.async_copy(ts_h, ts_all, sem)
    i1.wait()
    i2.wait()
    i3.wait()
    i4.wait()
    for k in range(CHUNK // 16):
        iv = idx_v[pl.ds(16 * k, 16)]
        wv = widx_v[pl.ds(16 * k, 16)]
        pidx_v[pl.ds(16 * k, 16)] = lax.shift_right_logical(iv, 1)
        lug_v[pl.ds(16 * k, 16)] = plsc.load_gather(lu_all, [iv])
        tsw_v[pl.ds(16 * k, 16)] = plsc.load_gather(ts_all, [wv])
    c1 = pltpu.async_copy(mem_h.at[idx_v], rows_v, sem)
    c2 = pltpu.async_copy(statp_h.at[pidx_v], stat_v, sem)
    c3 = pltpu.async_copy(msgp_h.at[widx_v], msg_v, sem)
    c1.wait()
    c2.wait()
    c3.wait()
    o1 = pltpu.async_copy(rows_v, rows_o.at[pl.ds(base, CHUNK)], sem)
    o2 = pltpu.async_copy(stat_v, statp_o.at[pl.ds(base, CHUNK)], sem)
    o3 = pltpu.async_copy(msg_v, msgw_o.at[pl.ds(base, CHUNK)], sem)
    o4 = pltpu.async_copy(lug_v, lug_o.at[pl.ds(base, CHUNK)], sem)
    o5 = pltpu.async_copy(tsw_v, tsw_o.at[pl.ds(base, CHUNK)], sem)
    o1.wait()
    o2.wait()
    o3.wait()
    o4.wait()
    o5.wait()


def _make_gather(interpret=False):
    mesh = plsc.VectorSubcoreMesh(
        core_axis_name="c", subcore_axis_name="s",
        num_cores=NC, num_subcores=NS)
    return functools.partial(
        pl.kernel,
        out_type=(
            jax.ShapeDtypeStruct((U, 2 * D), jnp.float32),
            jax.ShapeDtypeStruct((U, 2 * D), jnp.float32),
            jax.ShapeDtypeStruct((U, 2 * D), jnp.float32),
            jax.ShapeDtypeStruct((U,), jnp.float32),
            jax.ShapeDtypeStruct((U,), jnp.float32),
        ),
        mesh=mesh,
        compiler_params=pltpu.CompilerParams(needs_layout_passes=False),
        scratch_types=[
            pltpu.VMEM((CHUNK,), jnp.int32),
            pltpu.VMEM((CHUNK,), jnp.int32),
            pltpu.VMEM((CHUNK,), jnp.int32),
            pltpu.VMEM((CHUNK, 2 * D), jnp.float32),
            pltpu.VMEM((CHUNK, 2 * D), jnp.float32),
            pltpu.VMEM((CHUNK, 2 * D), jnp.float32),
            pltpu.VMEM((N,), jnp.float32),
            pltpu.VMEM((U,), jnp.float32),
            pltpu.VMEM((CHUNK,), jnp.float32),
            pltpu.VMEM((CHUNK,), jnp.float32),
            pltpu.SemaphoreType.DMA,
        ],
        interpret=interpret,
    )(_gather_body)


def _mlp(x, w1, b1, w2, b2):
    h = jnp.dot(x, w1, preferred_element_type=jnp.float32) + b1
    h = jnp.where(h >= 0, h, 0.01 * h)
    h = jnp.dot(h, w2, preferred_element_type=jnp.float32) + b2
    return jnp.where(h >= 0, h, 0.01 * h)


BLK_F = 1024


def _fix_body(rows_ref, lug_ref, tsw_ref, msgw_ref, statp_ref, src_ref,
              w1_ref, b1_ref, w2_ref, b2_ref, el_ref, nt_ref, fix_ref):
    rows = rows_ref[...]
    tsw = tsw_ref[...]
    dec = jnp.exp((lug_ref[...] - tsw) * INV30)
    msg = msgw_ref[...]
    x = rows * dec
    upd = jnp.concatenate(
        [x[:, :D] + msg[:, :D], x[:, D:] + msg[:, D:D + 1]], axis=1)
    h = _mlp(upd, w1_ref[...], b1_ref[...], w2_ref[...], b2_ref[...])
    outdec = jnp.exp((tsw - nt_ref[...]) * INV30)
    statp = statp_ref[...]
    odd = (src_ref[...] & 1) == 1
    statg = jnp.where(odd, statp[:, D:], statp[:, :D])
    el = el_ref[...]
    fix = el * statg + (1.0 - el) * outdec * h
    fix_ref[...] = jnp.concatenate([fix, fix], axis=1)


def _make_fix(interpret=False):
    const = lambda i: (0, 0)
    return pl.pallas_call(
        _fix_body,
        grid=(U // BLK_F,),
        in_specs=[
            pl.BlockSpec((BLK_F, 2 * D), lambda i: (i, 0)),
            pl.BlockSpec((BLK_F, 1), lambda i: (i, 0)),
            pl.BlockSpec((BLK_F, 1), lambda i: (i, 0)),
            pl.BlockSpec((BLK_F, 2 * D), lambda i: (i, 0)),
            pl.BlockSpec((BLK_F, 2 * D), lambda i: (i, 0)),
            pl.BlockSpec((BLK_F, 1), lambda i: (i, 0)),
            pl.BlockSpec((2 * D, D), const),
            pl.BlockSpec((1, D), const),
            pl.BlockSpec((D, D), const),
            pl.BlockSpec((1, D), const),
            pl.BlockSpec((1, 1), const),
            pl.BlockSpec((1, 1), const),
        ],
        out_specs=pl.BlockSpec((BLK_F, 2 * D), lambda i: (i, 0)),
        out_shape=jax.ShapeDtypeStruct((U, 2 * D), jnp.float32),
        interpret=interpret,
    )


BLK_N = 1000


def _dense_body(mem_ref, lu_ref, stat_ref,
                w1_ref, b1_ref, w2_ref, b2_ref, el_ref, nt_ref, out_ref):
    h = _mlp(mem_ref[...], w1_ref[...], b1_ref[...], w2_ref[...], b2_ref[...])
    outdec = jnp.exp((lu_ref[...] - nt_ref[...]) * INV30)
    el = el_ref[...]
    res = el * stat_ref[...] + (1.0 - el) * outdec * h
    out_ref[...] = jnp.concatenate([res, res], axis=1)


def _make_dense(interpret=False):
    const = lambda i: (0, 0)
    return pl.pallas_call(
        _dense_body,
        grid=(N // BLK_N,),
        in_specs=[
            pl.BlockSpec((BLK_N, 2 * D), lambda i: (i, 0)),
            pl.BlockSpec((BLK_N, 1), lambda i: (i, 0)),
            pl.BlockSpec((BLK_N, D), lambda i: (i, 0)),
            pl.BlockSpec((2 * D, D), const),
            pl.BlockSpec((1, D), const),
            pl.BlockSpec((D, D), const),
            pl.BlockSpec((1, D), const),
            pl.BlockSpec((1, 1), const),
            pl.BlockSpec((1, 1), const),
        ],
        out_specs=pl.BlockSpec((BLK_N, 2 * D), lambda i: (i, 0)),
        out_shape=jax.ShapeDtypeStruct((N, 2 * D), jnp.float32),
        interpret=interpret,
    )


def _scatter_body(fixw_h, sidx_h, final_ref, fix_v, idx_v, sem):
    wid = lax.axis_index("s") * NC + lax.axis_index("c")
    pltpu.sync_copy(fixw_h.at[pl.ds(wid * CHUNK, CHUNK)], fix_v)
    pltpu.sync_copy(sidx_h.at[wid], idx_v)
    pltpu.async_copy(fix_v, final_ref.at[idx_v.at[0]], sem).wait()


def _make_scatter(interpret=False):
    mesh = plsc.VectorSubcoreMesh(
        core_axis_name="c", subcore_axis_name="s",
        num_cores=NC, num_subcores=NS)
    return functools.partial(
        pl.kernel,
        out_type=(),
        mesh=mesh,
        scratch_types=[
            pltpu.VMEM((CHUNK, 2 * D), jnp.float32),
            pltpu.VMEM((1, 128), jnp.int32),
            pltpu.SemaphoreType.DMA,
        ],
        interpret=interpret,
    )(_scatter_body)


_winner = _make_winner()
_gather = _make_gather()
_fix = _make_fix()
_dense = _make_dense()
_scatter = _make_scatter()


def kernel(source_nodes, now_time, unique_sources, unique_messages,
           unique_timestamps, memory, last_update, W1, b1, W2, b2,
           static_emb, e_lamb):
    src = unique_sources.astype(jnp.int32)
    ts = unique_timestamps.astype(jnp.float32)
    nt = jnp.asarray(now_time, jnp.float32).reshape(1, 1)
    el = e_lamb.reshape(1, 1)
    lu2d = last_update.reshape(N, 1)
    b1r = b1.reshape(1, D)
    b2r = b2.reshape(1, D)
    statp_view = static_emb.reshape(N // 2, 2 * D)
    msgp = jnp.pad(unique_messages, ((0, 0), (0, 2 * D - (D + 1))))

    w = _winner(src.reshape(U, 1), src.reshape(1, U))
    rows_g, statp_g, msgw_g, lug, tsw = _gather(
        src, w.reshape(U), memory, statp_view, msgp, last_update, ts)
    fixw = _fix(rows_g, lug.reshape(U, 1), tsw.reshape(U, 1), msgw_g,
                statp_g, src.reshape(U, 1), W1, b1r, W2, b2r, el, nt)
    dense_w = _dense(memory, lu2d, static_emb, W1, b1r, W2, b2r, el, nt)
    final_ref = jax.new_ref(dense_w)
    _scatter(fixw, src.reshape(NW, 1, 128), final_ref)
    return final_ref[...][:, :D]
